# Initial kernel scaffold; baseline (speedup 1.0000x reference)
import jax
import jax.numpy as jnp
from jax.experimental import pallas as pl


def kernel(protein_1d_data, embedding_table):
    raise NotImplementedError("write your pallas kernel here")


# speedup vs baseline: 1.8007x; 1.8007x over previous
import functools

import jax
import jax.numpy as jnp
from jax import lax
from jax.experimental import pallas as pl
from jax.experimental.pallas import tpu as pltpu
from jax.experimental.pallas import tpu_sc as plsc

NC, NS = 2, 16
NW = NC * NS
D = 32
CHUNK = 128
K = 16
GROUP = CHUNK * K

BATCH = 16384
SEQ = 200
N = BATCH * SEQ
ROWS_PER_W = N // NW
GROUPS_PER_W = ROWS_PER_W // GROUP
IDX_ROWS_PER_W = ROWS_PER_W // CHUNK

_mesh = plsc.VectorSubcoreMesh(core_axis_name="c", subcore_axis_name="s")


@functools.partial(
    pl.kernel,
    out_type=jax.ShapeDtypeStruct((N, D), jnp.float32),
    mesh=_mesh,
    scratch_types=[
        pltpu.VMEM((K, CHUNK), jnp.int32),
        pltpu.VMEM((GROUP, D), jnp.float32),
        pltpu.SemaphoreType.DMA,
    ],
    compiler_params=pltpu.CompilerParams(use_tc_tiling_on_sc=False),
)
def _gather_kernel(idx_hbm, table_hbm, out_hbm, idx_v, rows_v, sem):
    wid = lax.axis_index("s") * NC + lax.axis_index("c")

    def group(g, carry):
        row0 = wid * IDX_ROWS_PER_W + g * K
        pltpu.sync_copy(idx_hbm.at[pl.ds(row0, K)], idx_v)
        copies = [
            pltpu.make_async_copy(
                table_hbm.at[idx_v.at[j]],
                rows_v.at[pl.ds(j * CHUNK, CHUNK)],
                sem,
            )
            for j in range(K)
        ]
        for cp in copies:
            cp.start()
        for cp in copies:
            cp.wait()
        out0 = wid * ROWS_PER_W + g * GROUP
        pltpu.sync_copy(rows_v, out_hbm.at[pl.ds(out0, GROUP)])
        return carry

    lax.fori_loop(0, GROUPS_PER_W, group, 0)


def kernel(protein_1d_data, embedding_table):
    idx = protein_1d_data.astype(jnp.int32).reshape(N // CHUNK, CHUNK)
    out = _gather_kernel(idx, embedding_table)
    return out.reshape(BATCH, SEQ, D)


# speedup vs baseline: 6.9488x; 3.8588x over previous
import functools

import jax
import jax.numpy as jnp
from jax import lax
from jax.experimental import pallas as pl
from jax.experimental.pallas import tpu as pltpu
from jax.experimental.pallas import tpu_sc as plsc

NC, NS = 2, 16
NW = NC * NS
D = 32
VOCAB = 26
CHUNK = 128
K = 8
GROUP = CHUNK * K

BATCH = 16384
SEQ = 200
N = BATCH * SEQ
ROWS_PER_W = N // NW
GROUPS_PER_W = ROWS_PER_W // GROUP
IDX_ROWS_PER_W = ROWS_PER_W // CHUNK

_mesh = plsc.VectorSubcoreMesh(core_axis_name="c", subcore_axis_name="s")


@functools.partial(
    pl.kernel,
    out_type=jax.ShapeDtypeStruct((N, D), jnp.float32),
    mesh=_mesh,
    scratch_types=[
        pltpu.VMEM_SHARED((VOCAB, D), jnp.float32),
        pltpu.VMEM((2 * K, CHUNK), jnp.int32),
        pltpu.VMEM((2 * GROUP, D), jnp.float32),
        pltpu.SemaphoreType.DMA,
        pltpu.SemaphoreType.DMA,
        pltpu.SemaphoreType.DMA,
        pltpu.SemaphoreType.DMA,
        pltpu.SemaphoreType.DMA,
    ],
    compiler_params=pltpu.CompilerParams(use_tc_tiling_on_sc=False),
)
def _gather_kernel(idx_hbm, table_hbm, out_hbm, table_v, idx_v, rows_v,
                   gat_sem, i_sem0, i_sem1, o_sem0, o_sem1):
    wid = lax.axis_index("s") * NC + lax.axis_index("c")
    idx0 = wid * IDX_ROWS_PER_W
    out0 = wid * ROWS_PER_W
    i_sems = (i_sem0, i_sem1)
    o_sems = (o_sem0, o_sem1)

    @pl.when(lax.axis_index("s") == 0)
    def _stage_table():
        pltpu.sync_copy(table_hbm, table_v)

    plsc.subcore_barrier()

    def idx_copy(g, buf):
        return pltpu.make_async_copy(
            idx_hbm.at[pl.ds(idx0 + g * K, K)],
            idx_v.at[pl.ds(buf * K, K)],
            i_sems[buf])

    def out_copy(g, buf):
        return pltpu.make_async_copy(
            rows_v.at[pl.ds(buf * GROUP, GROUP)],
            out_hbm.at[pl.ds(out0 + g * GROUP, GROUP)],
            o_sems[buf])

    def phase(g, buf):
        @pl.when(g + 1 < GROUPS_PER_W)
        def _prefetch():
            idx_copy(g + 1, 1 - buf).start()

        idx_copy(g, buf).wait()

        @pl.when(g >= 2)
        def _reuse():
            out_copy(g - 2, buf).wait()

        copies = [
            pltpu.make_async_copy(
                table_v.at[idx_v.at[buf * K + j]],
                rows_v.at[pl.ds(buf * GROUP + j * CHUNK, CHUNK)],
                gat_sem,
            )
            for j in range(K)
        ]
        for cp in copies:
            cp.start()
        for cp in copies:
            cp.wait()

        out_copy(g, buf).start()

    idx_copy(0, 0).start()

    def pair(t, carry):
        phase(2 * t, 0)
        phase(2 * t + 1, 1)
        return carry

    lax.fori_loop(0, GROUPS_PER_W // 2, pair, 0)

    out_copy(GROUPS_PER_W - 2, 0).wait()
    out_copy(GROUPS_PER_W - 1, 1).wait()


def kernel(protein_1d_data, embedding_table):
    idx = protein_1d_data.astype(jnp.int32).reshape(N // CHUNK, CHUNK)
    out = _gather_kernel(idx, embedding_table)
    return out.reshape(BATCH, SEQ, D)
